# baseline (speedup 1.0000x reference)
import functools

import jax
import jax.numpy as jnp
from jax import lax
from jax.experimental import pallas as pl
from jax.experimental.pallas import tpu as pltpu
from jax.experimental.pallas import tpu_sc as plsc

N = 10000
D = 128
H = 64
NG = 256

_NC, _NS = 2, 16
_NW = _NC * _NS
_NP = 10112
_ZR = _NP // _NS
_EC = 128
_ER = 2560
_RPT = _ER // _NW
_BLK = 2528

_CPS = 2
_NCH = _RPT // _CPS
_ZT = _NP - N

_mesh = plsc.VectorSubcoreMesh(core_axis_name="c", subcore_axis_name="s")


@functools.partial(
    pl.kernel,
    out_type=jax.ShapeDtypeStruct((_NC, _NP, H), jnp.float32),
    mesh=_mesh,
    scratch_types=[
        pltpu.VMEM((_NCH, _CPS * _EC), jnp.int32),
        pltpu.VMEM((2, _CPS * _EC), jnp.int32),
        pltpu.VMEM((2, _CPS * _EC, H), jnp.float32),
        pltpu.VMEM_SHARED((_NP, H), jnp.float32),
        pltpu.VMEM_SHARED((_NP, H), jnp.float32),
        pltpu.SemaphoreType.DMA,
        pltpu.SemaphoreType.DMA,
        pltpu.SemaphoreType.DMA,
        pltpu.SemaphoreType.DMA,
    ],
    compiler_params=pltpu.CompilerParams(use_tc_tiling_on_sc=False),
)
def _sc_agg(y_hbm, src_hbm, dst_hbm, out_hbm, srcv, dring, rows,
            accum, ycopy, gsem, ssem, dsem, csem):
    cid = lax.axis_index("c")
    sid = lax.axis_index("s")
    wid = sid * _NC + cid

    pltpu.async_copy(y_hbm.at[pl.ds(sid * _ZR, _ZR)],
                     ycopy.at[pl.ds(sid * _ZR, _ZR)], ssem)

    _rem = _ZR - (_ZR // _ZT) * _ZT
    for q in range(_ZR // _ZT):
        pltpu.async_copy(y_hbm.at[pl.ds(N, _ZT)],
                         accum.at[pl.ds(sid * _ZR + q * _ZT, _ZT)], csem)
    if _rem:
        pltpu.async_copy(y_hbm.at[pl.ds(N, _rem)],
                         accum.at[pl.ds(sid * _ZR + (_ZR // _ZT) * _ZT, _rem)],
                         csem)

    pltpu.sync_copy(src_hbm.at[wid], srcv)
    pltpu.async_copy(dst_hbm.at[wid, 0], dring.at[0], dsem)
    for q in range(_ZR // _ZT):
        pltpu.make_async_copy(y_hbm.at[pl.ds(N, _ZT)],
                              accum.at[pl.ds(sid * _ZR + q * _ZT, _ZT)],
                              csem).wait()
    if _rem:
        pltpu.make_async_copy(
            y_hbm.at[pl.ds(N, _rem)],
            accum.at[pl.ds(sid * _ZR + (_ZR // _ZT) * _ZT, _rem)],
            csem).wait()
    pltpu.make_async_copy(y_hbm.at[pl.ds(sid * _ZR, _ZR)],
                          ycopy.at[pl.ds(sid * _ZR, _ZR)], ssem).wait()
    plsc.subcore_barrier()

    pltpu.async_copy(ycopy.at[srcv.at[0]], rows.at[0], gsem)

    def _outer(i, carry):
        for b in range(2):
            g = i * 2 + b
            nxt = g + 1

            pltpu.make_async_copy(ycopy.at[pl.ds(0, _CPS * _EC)],
                                  rows.at[b], gsem).wait()

            @pl.when(nxt < _NCH)
            def _():
                pltpu.async_copy(ycopy.at[srcv.at[nxt]], rows.at[1 - b],
                                 gsem)
                pltpu.async_copy(dst_hbm.at[wid, nxt], dring.at[1 - b],
                                 dsem)

            pltpu.make_async_copy(dst_hbm.at[wid, 0], dring.at[b],
                                  dsem).wait()
            pltpu.sync_copy(rows.at[b], accum.at[dring.at[b]], add=True)
        return carry

    lax.fori_loop(0, _NCH // 2, _outer, 0)

    plsc.subcore_barrier()
    pltpu.sync_copy(accum.at[pl.ds(sid * _ZR, _ZR)],
                    out_hbm.at[cid, pl.ds(sid * _ZR, _ZR)])


def _dot(a, b):
    return jnp.dot(a, b, preferred_element_type=jnp.float32)


def _tc_first_body(x_ref, w_ref, o_ref):
    o_ref[...] = _dot(x_ref[...], w_ref[...])


def _tc_first(x_pad, w):
    return pl.pallas_call(
        _tc_first_body,
        grid=(1,),
        in_specs=[
            pl.BlockSpec((_NP, D), lambda i: (0, 0)),
            pl.BlockSpec((D, H), lambda i: (0, 0)),
        ],
        out_specs=pl.BlockSpec((_NP, H), lambda i: (0, 0)),
        out_shape=jax.ShapeDtypeStruct((_NP, H), jnp.float32),
    )(x_pad, w)


def _row_mask(i, v, blk):
    rowid = i * blk + lax.broadcasted_iota(jnp.int32, (blk, 1), 0)
    return jnp.where(rowid < N, v, 0.0)


def _tc_mid_body(y_ref, p_ref, b1_ref, w2_ref, b2_ref, w1n_ref, o_ref):
    h = jnp.maximum(y_ref[...] + p_ref[0] + p_ref[1] + b1_ref[...], 0.0)
    h = _dot(h, w2_ref[...]) + b2_ref[...]
    x = jnp.maximum(h, 0.0)
    o_ref[...] = _row_mask(0, _dot(x, w1n_ref[...]), _NP)


def _tc_mid(y, p, b1, w2, b2, w1n):
    return pl.pallas_call(
        _tc_mid_body,
        grid=(1,),
        in_specs=[
            pl.BlockSpec((_NP, H), lambda i: (0, 0)),
            pl.BlockSpec((_NC, _NP, H), lambda i: (0, 0, 0)),
            pl.BlockSpec((1, H), lambda i: (0, 0)),
            pl.BlockSpec((H, H), lambda i: (0, 0)),
            pl.BlockSpec((1, H), lambda i: (0, 0)),
            pl.BlockSpec((H, H), lambda i: (0, 0)),
        ],
        out_specs=pl.BlockSpec((_NP, H), lambda i: (0, 0)),
        out_shape=jax.ShapeDtypeStruct((_NP, H), jnp.float32),
    )(y, p, b1, w2, b2, w1n)


def _tc_final_body(y_ref, p_ref, b1_ref, w2_ref, b2_ref, batch_ref,
                   mw1_ref, mb1_ref, mw2_ref, mb2_ref, o_ref, g_acc):
    i = pl.program_id(0)

    @pl.when(i == 0)
    def _():
        g_acc[...] = jnp.zeros_like(g_acc)

    h = jnp.maximum(y_ref[...] + p_ref[0] + p_ref[1] + b1_ref[...], 0.0)
    h = _dot(h, w2_ref[...]) + b2_ref[...]
    x = _row_mask(i, jnp.maximum(h, 0.0), _BLK)
    b = batch_ref[0, 0]
    oh = (b[:, None] == lax.broadcasted_iota(jnp.int32, (_BLK, NG), 1)
          ).astype(jnp.float32)
    g_acc[...] += lax.dot_general(
        oh, x, (((0,), (0,)), ((), ())),
        preferred_element_type=jnp.float32)

    @pl.when(i == _NP // _BLK - 1)
    def _():
        g = jnp.maximum(_dot(g_acc[...], mw1_ref[...]) + mb1_ref[...], 0.0)
        o_ref[...] = _dot(g, mw2_ref[...]) + mb2_ref[...]


def _tc_final(y, p, b1, w2, b2, batch3, mw1, mb1, mw2, mb2):
    return pl.pallas_call(
        _tc_final_body,
        grid=(_NP // _BLK,),
        in_specs=[
            pl.BlockSpec((_BLK, H), lambda i: (i, 0)),
            pl.BlockSpec((_NC, _BLK, H), lambda i: (0, i, 0)),
            pl.BlockSpec((1, H), lambda i: (0, 0)),
            pl.BlockSpec((H, H), lambda i: (0, 0)),
            pl.BlockSpec((1, H), lambda i: (0, 0)),
            pl.BlockSpec((1, 1, _BLK), lambda i: (i, 0, 0)),
            pl.BlockSpec((H, H), lambda i: (0, 0)),
            pl.BlockSpec((1, H), lambda i: (0, 0)),
            pl.BlockSpec((H, H), lambda i: (0, 0)),
            pl.BlockSpec((1, H), lambda i: (0, 0)),
        ],
        out_specs=pl.BlockSpec((NG, H), lambda i: (0, 0)),
        out_shape=jax.ShapeDtypeStruct((NG, H), jnp.float32),
        scratch_shapes=[pltpu.VMEM((NG, H), jnp.float32)],
    )(y, p, b1, w2, b2, batch3, mw1, mb1, mw2, mb2)


def kernel(x, edge_index, batch,
           w1_0, b1_0, w2_0, b2_0, w1_1, b1_1, w2_1, b2_1,
           w1_2, b1_2, w2_2, b2_2, w1_3, b1_3, w2_3, b2_3,
           w1_4, b1_4, w2_4, b2_4, mw1, mb1, mw2, mb2):
    conv = [(w1_0, b1_0, w2_0, b2_0), (w1_1, b1_1, w2_1, b2_1),
            (w1_2, b1_2, w2_2, b2_2), (w1_3, b1_3, w2_3, b2_3),
            (w1_4, b1_4, w2_4, b2_4)]

    epad = _ER * _EC - edge_index.shape[1]
    fill = jnp.full((epad,), N, jnp.int32)
    srcp = jnp.concatenate([edge_index[0], fill]).reshape(
        _NW, _NCH, _CPS * _EC)
    dstp = jnp.concatenate([edge_index[1], fill]).reshape(
        _NW, _NCH, _CPS * _EC)
    x_pad = jnp.zeros((_NP, D), jnp.float32).at[:N].set(x)
    batch3 = jnp.concatenate(
        [batch, jnp.zeros((_NP - N,), jnp.int32)]).reshape(_NP // _BLK, 1, _BLK)

    y = _tc_first(x_pad, w1_0)
    for i in range(5):
        _, b1, w2, b2 = conv[i]
        p = _sc_agg(y, srcp, dstp)
        b1r = b1.reshape(1, H)
        b2r = b2.reshape(1, H)
        if i < 4:
            y = _tc_mid(y, p, b1r, w2, b2r, conv[i + 1][0])
        else:
            out = _tc_final(y, p, b1r, w2, b2r, batch3,
                            mw1, mb1.reshape(1, H), mw2, mb2.reshape(1, H))
    return out


# speedup vs baseline: 1.0025x; 1.0025x over previous
import functools

import jax
import jax.numpy as jnp
from jax import lax
from jax.experimental import pallas as pl
from jax.experimental.pallas import tpu as pltpu
from jax.experimental.pallas import tpu_sc as plsc

N = 10000
D = 128
H = 64
NG = 256

_NC, _NS = 2, 16
_NW = _NC * _NS
_NP = 10112
_ZR = _NP // _NS
_EC = 128
_ER = 2560
_RPT = _ER // _NW
_BLK = 2528

_CPS = 2
_NCH = _RPT // _CPS
_ZT = _NP - N

_mesh = plsc.VectorSubcoreMesh(core_axis_name="c", subcore_axis_name="s")


@functools.partial(
    pl.kernel,
    out_type=jax.ShapeDtypeStruct((_NC, _NP, H), jnp.float32),
    mesh=_mesh,
    scratch_types=[
        pltpu.VMEM((_NCH, _CPS * _EC), jnp.int32),
        pltpu.VMEM((2, _CPS * _EC), jnp.int32),
        pltpu.VMEM((2, _CPS * _EC, H), jnp.float32),
        pltpu.VMEM_SHARED((_NP, H), jnp.float32),
        pltpu.VMEM_SHARED((_NP, H), jnp.float32),
        pltpu.SemaphoreType.DMA,
        pltpu.SemaphoreType.DMA,
        pltpu.SemaphoreType.DMA,
        pltpu.SemaphoreType.DMA,
    ],
    compiler_params=pltpu.CompilerParams(use_tc_tiling_on_sc=False),
)
def _sc_agg(y_hbm, src_hbm, dst_hbm, out_hbm, srcv, dring, rows,
            accum, ycopy, gsem, ssem, dsem, csem):
    cid = lax.axis_index("c")
    sid = lax.axis_index("s")
    wid = sid * _NC + cid

    pltpu.async_copy(y_hbm.at[pl.ds(sid * _ZR, _ZR)],
                     ycopy.at[pl.ds(sid * _ZR, _ZR)], ssem)

    _rem = _ZR - (_ZR // _ZT) * _ZT
    for q in range(_ZR // _ZT):
        pltpu.async_copy(y_hbm.at[pl.ds(N, _ZT)],
                         accum.at[pl.ds(sid * _ZR + q * _ZT, _ZT)], csem)
    if _rem:
        pltpu.async_copy(y_hbm.at[pl.ds(N, _rem)],
                         accum.at[pl.ds(sid * _ZR + (_ZR // _ZT) * _ZT, _rem)],
                         csem)

    pltpu.sync_copy(src_hbm.at[wid], srcv)
    pltpu.async_copy(dst_hbm.at[wid, 0], dring.at[0], dsem)
    for q in range(_ZR // _ZT):
        pltpu.make_async_copy(y_hbm.at[pl.ds(N, _ZT)],
                              accum.at[pl.ds(sid * _ZR + q * _ZT, _ZT)],
                              csem).wait()
    if _rem:
        pltpu.make_async_copy(
            y_hbm.at[pl.ds(N, _rem)],
            accum.at[pl.ds(sid * _ZR + (_ZR // _ZT) * _ZT, _rem)],
            csem).wait()
    pltpu.make_async_copy(y_hbm.at[pl.ds(sid * _ZR, _ZR)],
                          ycopy.at[pl.ds(sid * _ZR, _ZR)], ssem).wait()
    plsc.subcore_barrier()

    pltpu.async_copy(ycopy.at[srcv.at[0]], rows.at[0], gsem)

    def _outer(i, carry):
        for b in range(2):
            g = i * 2 + b
            nxt = g + 1

            pltpu.make_async_copy(ycopy.at[pl.ds(0, _CPS * _EC)],
                                  rows.at[b], gsem).wait()

            @pl.when(nxt < _NCH)
            def _():
                pltpu.async_copy(ycopy.at[srcv.at[nxt]], rows.at[1 - b],
                                 gsem)
                pltpu.async_copy(dst_hbm.at[wid, nxt], dring.at[1 - b],
                                 dsem)

            pltpu.make_async_copy(dst_hbm.at[wid, 0], dring.at[b],
                                  dsem).wait()
            pltpu.sync_copy(rows.at[b], accum.at[dring.at[b]], add=True)
        return carry

    lax.fori_loop(0, _NCH // 2, _outer, 0)

    plsc.subcore_barrier()
    pltpu.sync_copy(accum.at[pl.ds(sid * _ZR, _ZR)],
                    out_hbm.at[cid, pl.ds(sid * _ZR, _ZR)])


def _dot(a, b):
    return jnp.dot(a, b, preferred_element_type=jnp.float32)


def _tc_first_body(x_ref, w_ref, o_ref):
    o_ref[...] = _dot(x_ref[...], w_ref[...])


def _tc_first(x_pad, w):
    return pl.pallas_call(
        _tc_first_body,
        grid=(_NP // _BLK,),
        in_specs=[
            pl.BlockSpec((_BLK, D), lambda i: (i, 0)),
            pl.BlockSpec((D, H), lambda i: (0, 0)),
        ],
        out_specs=pl.BlockSpec((_BLK, H), lambda i: (i, 0)),
        out_shape=jax.ShapeDtypeStruct((_NP, H), jnp.float32),
    )(x_pad, w)


def _row_mask(i, v, blk):
    rowid = i * blk + lax.broadcasted_iota(jnp.int32, (blk, 1), 0)
    return jnp.where(rowid < N, v, 0.0)


def _tc_mid_body(y_ref, p_ref, b1_ref, w2_ref, b2_ref, w1n_ref, o_ref):
    i = pl.program_id(0)
    h = jnp.maximum(y_ref[...] + p_ref[0] + p_ref[1] + b1_ref[...], 0.0)
    h = _dot(h, w2_ref[...]) + b2_ref[...]
    x = jnp.maximum(h, 0.0)
    o_ref[...] = _row_mask(i, _dot(x, w1n_ref[...]), _BLK)


def _tc_mid(y, p, b1, w2, b2, w1n):
    return pl.pallas_call(
        _tc_mid_body,
        grid=(_NP // _BLK,),
        in_specs=[
            pl.BlockSpec((_BLK, H), lambda i: (i, 0)),
            pl.BlockSpec((_NC, _BLK, H), lambda i: (0, i, 0)),
            pl.BlockSpec((1, H), lambda i: (0, 0)),
            pl.BlockSpec((H, H), lambda i: (0, 0)),
            pl.BlockSpec((1, H), lambda i: (0, 0)),
            pl.BlockSpec((H, H), lambda i: (0, 0)),
        ],
        out_specs=pl.BlockSpec((_BLK, H), lambda i: (i, 0)),
        out_shape=jax.ShapeDtypeStruct((_NP, H), jnp.float32),
    )(y, p, b1, w2, b2, w1n)


def _tc_final_body(y_ref, p_ref, b1_ref, w2_ref, b2_ref, batch_ref,
                   mw1_ref, mb1_ref, mw2_ref, mb2_ref, o_ref, g_acc):
    i = pl.program_id(0)

    @pl.when(i == 0)
    def _():
        g_acc[...] = jnp.zeros_like(g_acc)

    h = jnp.maximum(y_ref[...] + p_ref[0] + p_ref[1] + b1_ref[...], 0.0)
    h = _dot(h, w2_ref[...]) + b2_ref[...]
    x = _row_mask(i, jnp.maximum(h, 0.0), _BLK)
    b = batch_ref[0, 0]
    oh = (b[:, None] == lax.broadcasted_iota(jnp.int32, (_BLK, NG), 1)
          ).astype(jnp.float32)
    g_acc[...] += lax.dot_general(
        oh, x, (((0,), (0,)), ((), ())),
        preferred_element_type=jnp.float32)

    @pl.when(i == _NP // _BLK - 1)
    def _():
        g = jnp.maximum(_dot(g_acc[...], mw1_ref[...]) + mb1_ref[...], 0.0)
        o_ref[...] = _dot(g, mw2_ref[...]) + mb2_ref[...]


def _tc_final(y, p, b1, w2, b2, batch3, mw1, mb1, mw2, mb2):
    return pl.pallas_call(
        _tc_final_body,
        grid=(_NP // _BLK,),
        in_specs=[
            pl.BlockSpec((_BLK, H), lambda i: (i, 0)),
            pl.BlockSpec((_NC, _BLK, H), lambda i: (0, i, 0)),
            pl.BlockSpec((1, H), lambda i: (0, 0)),
            pl.BlockSpec((H, H), lambda i: (0, 0)),
            pl.BlockSpec((1, H), lambda i: (0, 0)),
            pl.BlockSpec((1, 1, _BLK), lambda i: (i, 0, 0)),
            pl.BlockSpec((H, H), lambda i: (0, 0)),
            pl.BlockSpec((1, H), lambda i: (0, 0)),
            pl.BlockSpec((H, H), lambda i: (0, 0)),
            pl.BlockSpec((1, H), lambda i: (0, 0)),
        ],
        out_specs=pl.BlockSpec((NG, H), lambda i: (0, 0)),
        out_shape=jax.ShapeDtypeStruct((NG, H), jnp.float32),
        scratch_shapes=[pltpu.VMEM((NG, H), jnp.float32)],
    )(y, p, b1, w2, b2, batch3, mw1, mb1, mw2, mb2)


def kernel(x, edge_index, batch,
           w1_0, b1_0, w2_0, b2_0, w1_1, b1_1, w2_1, b2_1,
           w1_2, b1_2, w2_2, b2_2, w1_3, b1_3, w2_3, b2_3,
           w1_4, b1_4, w2_4, b2_4, mw1, mb1, mw2, mb2):
    conv = [(w1_0, b1_0, w2_0, b2_0), (w1_1, b1_1, w2_1, b2_1),
            (w1_2, b1_2, w2_2, b2_2), (w1_3, b1_3, w2_3, b2_3),
            (w1_4, b1_4, w2_4, b2_4)]

    epad = _ER * _EC - edge_index.shape[1]
    fill = jnp.full((epad,), N, jnp.int32)
    srcp = jnp.concatenate([edge_index[0], fill]).reshape(
        _NW, _NCH, _CPS * _EC)
    dstp = jnp.concatenate([edge_index[1], fill]).reshape(
        _NW, _NCH, _CPS * _EC)
    x_pad = jnp.zeros((_NP, D), jnp.float32).at[:N].set(x)
    batch3 = jnp.concatenate(
        [batch, jnp.zeros((_NP - N,), jnp.int32)]).reshape(_NP // _BLK, 1, _BLK)

    y = _tc_first(x_pad, w1_0)
    for i in range(5):
        _, b1, w2, b2 = conv[i]
        p = _sc_agg(y, srcp, dstp)
        b1r = b1.reshape(1, H)
        b2r = b2.reshape(1, H)
        if i < 4:
            y = _tc_mid(y, p, b1r, w2, b2r, conv[i + 1][0])
        else:
            out = _tc_final(y, p, b1r, w2, b2r, batch3,
                            mw1, mb1.reshape(1, H), mw2, mb2.reshape(1, H))
    return out
